# Initial kernel scaffold; baseline (speedup 1.0000x reference)
import jax
import jax.numpy as jnp
from jax.experimental import pallas as pl


def kernel(logits, targets):
    raise NotImplementedError("write your pallas kernel here")


# speedup vs baseline: 8.1529x; 8.1529x over previous
import functools

import jax
import jax.numpy as jnp
from jax import lax
from jax.experimental import pallas as pl
from jax.experimental.pallas import tpu as pltpu
from jax.experimental.pallas import tpu_sc as plsc

NC = 2
NS = 16
L = 16
C = 19
N = 4 * 384 * 384
HW = 384 * 384
M = 32768
PIX_PER_TILE = N // NS
CHUNK = 1024
NCHUNK = PIX_PER_TILE // CHUNK
VPC = CHUNK // L
MB = M // NS
CLS_PER_CORE = 10


def _body(logits_hbm, targets_hbm, out_hbm, s_hbm, m_hbm,
          buf19, lbuf, tbuf, sbuf, mbuf, hist, slotbuf, acnt, afg,
          commbuf, accbuf, slots_sh, comm_sh):
    ci = lax.axis_index("c")
    si = lax.axis_index("s")
    p_base = si * PIX_PER_TILE
    b = si // 4
    off_base = (si % 4) * PIX_PER_TILE

    lanes = lax.iota(jnp.int32, L)
    zf = jnp.zeros((L,), jnp.float32)
    nf = jnp.float32(N)

    def chunk_a(k, _):
        off = off_base + k * CHUNK
        for j in range(C):
            pltpu.sync_copy(
                logits_hbm.at[pl.ds((b * C + j) * HW + off, CHUNK)],
                buf19.at[pl.ds(j * CHUNK, CHUNK)])

        def vreg_a(v, _):
            x0 = buf19[pl.ds(v * L, L)]
            m = x0
            for j in range(1, C):
                m = jnp.maximum(m, buf19[pl.ds(j * CHUNK + v * L, L)])
            den = zf
            for j in range(C):
                den = den + jnp.exp(buf19[pl.ds(j * CHUNK + v * L, L)] - m)
            sbuf[pl.ds(v * L, L)] = 1.0 / den
            mbuf[pl.ds(v * L, L)] = m
            return 0

        lax.fori_loop(0, VPC, vreg_a, 0, unroll=2)
        p0 = p_base + k * CHUNK
        pltpu.sync_copy(sbuf, s_hbm.at[pl.ds(p0, CHUNK)])
        pltpu.sync_copy(mbuf, m_hbm.at[pl.ds(p0, CHUNK)])
        return 0

    lax.fori_loop(0, NCHUNK, chunk_a, 0)

    def class_step(ki, acc):
        c = jnp.where(ci == 0, ki, CLS_PER_CORE + ki)

        def zero_h(i, _):
            hist[pl.ds(i * L, L)] = jnp.zeros((L,), jnp.int32)
            return 0
        lax.fori_loop(0, M // L, zero_h, 0, unroll=4)

        def chunk_b(k, _):
            off = off_base + k * CHUNK
            p0 = p_base + k * CHUNK
            pltpu.sync_copy(
                logits_hbm.at[pl.ds((b * C + c) * HW + off, CHUNK)], lbuf)
            pltpu.sync_copy(targets_hbm.at[pl.ds(p0, CHUNK)], tbuf)
            pltpu.sync_copy(s_hbm.at[pl.ds(p0, CHUNK)], sbuf)
            pltpu.sync_copy(m_hbm.at[pl.ds(p0, CHUNK)], mbuf)

            def vreg_b(v, _):
                z = lbuf[pl.ds(v * L, L)] - mbuf[pl.ds(v * L, L)]
                p = jnp.exp(jnp.minimum(z, 1.0)) * sbuf[pl.ds(v * L, L)]
                fgm = tbuf[pl.ds(v * L, L)] == c
                e = jnp.abs(jnp.where(fgm, 1.0, 0.0) - p)
                bk = jnp.minimum((e * jnp.float32(M)).astype(jnp.int32), M - 1)
                val = jnp.where(fgm, 65537, 65536).astype(jnp.int32)
                plsc.addupdate_scatter(hist, [bk], val)
                return 0

            lax.fori_loop(0, VPC, vreg_b, 0, unroll=2)
            return 0

        lax.fori_loop(0, NCHUNK, chunk_b, 0)

        pltpu.sync_copy(hist, slots_sh.at[pl.ds(si * M, M)])
        plsc.subcore_barrier()

        base_bkt = si * MB

        def merge_tile(t, carry):
            pltpu.sync_copy(slots_sh.at[pl.ds(t * M + base_bkt, MB)], slotbuf)

            def vreg_m(v, cr):
                tc, tf = cr
                u = slotbuf[pl.ds(v * L, L)]
                cnt = (u >> 16) & 0xFFFF
                fgc = u & 0xFFFF
                acnt[pl.ds(v * L, L)] = (
                    jnp.where(t == 0, 0.0, acnt[pl.ds(v * L, L)])
                    + cnt.astype(jnp.float32))
                afg[pl.ds(v * L, L)] = (
                    jnp.where(t == 0, 0.0, afg[pl.ds(v * L, L)])
                    + fgc.astype(jnp.float32))
                return (tc + jnp.sum(cnt.astype(jnp.float32)),
                        tf + jnp.sum(fgc.astype(jnp.float32)))

            return lax.fori_loop(0, MB // L, vreg_m, carry, unroll=2)

        tot_c, tot_f = lax.fori_loop(
            0, NS, merge_tile, (jnp.float32(0.0), jnp.float32(0.0)))

        commbuf[...] = jnp.where(lanes == 0, tot_c,
                                 jnp.where(lanes == 1, tot_f, 0.0))
        pltpu.sync_copy(commbuf, comm_sh.at[pl.ds(si * L, L)])
        plsc.subcore_barrier()

        p_tot = jnp.float32(0.0)
        k_above = jnp.float32(0.0)
        f_above = jnp.float32(0.0)
        for t in range(NS):
            pltpu.sync_copy(comm_sh.at[pl.ds(t * L, L)], commbuf)
            row = commbuf[...]
            tc = jnp.sum(jnp.where(lanes == 0, row, 0.0))
            tf = jnp.sum(jnp.where(lanes == 1, row, 0.0))
            p_tot = p_tot + tf
            gt = jnp.where(t > si, 1.0, 0.0)
            k_above = k_above + gt * tc
            f_above = f_above + gt * tf

        def vreg_s(i, carry):
            kc, fc, ls = carry
            v = MB // L - 1 - i
            cnt = acnt[pl.ds(v * L, L)]
            fgc = afg[pl.ds(v * L, L)]
            tot_cv = jnp.sum(cnt)
            tot_fv = jnp.sum(fgc)
            kb = kc + tot_cv - plsc.cumsum(cnt)
            fb = fc + tot_fv - plsc.cumsum(fgc)
            bb = kb - fb
            j = base_bkt + v * L + lanes
            jf = j.astype(jnp.float32)
            w = (jf + 0.5) * jnp.float32(1.0 / M)
            wprev = jnp.where(j + 1 >= M, 0.0, (jf + 1.5) * jnp.float32(1.0 / M))
            den = jnp.maximum(p_tot + bb, 1.0)
            term = jnp.where(kb < nf, (w - wprev) * (kb + 1.0) / den, 0.0)
            return (kc + tot_cv, fc + tot_fv, ls + term)

        _, _, lsum = lax.fori_loop(
            0, MB // L, vreg_s, (k_above, f_above, zf), unroll=2)

        acc = acc + jnp.where(p_tot > 0.0, lsum, zf)
        plsc.subcore_barrier()
        return acc

    acc = lax.fori_loop(0, CLS_PER_CORE, class_step, zf)
    accbuf[...] = acc
    pltpu.sync_copy(accbuf, out_hbm.at[pl.ds((ci * NS + si) * L, L)])


@functools.partial(jax.jit, static_argnames=())
def kernel(logits, targets):
    logits1d = logits.reshape(4 * C * HW)
    targets1d = targets.reshape(N)
    mesh = plsc.VectorSubcoreMesh(
        core_axis_name="c", subcore_axis_name="s",
        num_cores=NC, num_subcores=NS)
    run = pl.kernel(
        _body,
        out_type=(jax.ShapeDtypeStruct((NC * NS * L,), jnp.float32),
                  jax.ShapeDtypeStruct((N,), jnp.float32),
                  jax.ShapeDtypeStruct((N,), jnp.float32)),
        mesh=mesh,
        compiler_params=pltpu.CompilerParams(needs_layout_passes=False),
        scratch_types=[
            pltpu.VMEM((C * CHUNK,), jnp.float32),
            pltpu.VMEM((CHUNK,), jnp.float32),
            pltpu.VMEM((CHUNK,), jnp.int32),
            pltpu.VMEM((CHUNK,), jnp.float32),
            pltpu.VMEM((CHUNK,), jnp.float32),
            pltpu.VMEM((M,), jnp.int32),
            pltpu.VMEM((MB,), jnp.int32),
            pltpu.VMEM((MB,), jnp.float32),
            pltpu.VMEM((MB,), jnp.float32),
            pltpu.VMEM((L,), jnp.float32),
            pltpu.VMEM((L,), jnp.float32),
            pltpu.VMEM_SHARED((NS * M,), jnp.int32),
            pltpu.VMEM_SHARED((NS * L,), jnp.float32),
        ],
    )
    partials, _, _ = run(logits1d, targets1d)
    return jnp.sum(partials)


# speedup vs baseline: 13.6170x; 1.6702x over previous
import functools

import jax
import jax.numpy as jnp
from jax import lax
from jax.experimental import pallas as pl
from jax.experimental.pallas import tpu as pltpu
from jax.experimental.pallas import tpu_sc as plsc

NC = 2
NS = 16
L = 16
C = 19
N = 4 * 384 * 384
HW = 384 * 384
M = 32768
PIX_PER_TILE = N // NS
CHUNK = 1024
NCHUNK = PIX_PER_TILE // CHUNK
VPC = CHUNK // L
MB = M // NS
CLS_PER_CORE = 10


def _body(logits_hbm, targets_hbm, out_hbm, s_hbm, m_hbm,
          buf19, lbuf, tbuf, sbuf, mbuf, hist, slot16, acnt, afg,
          commbuf, accbuf, sem, slots_sh, comm_sh):
    ci = lax.axis_index("c")
    si = lax.axis_index("s")
    p_base = si * PIX_PER_TILE
    b = si // 4
    off_base = (si % 4) * PIX_PER_TILE

    lanes = lax.iota(jnp.int32, L)
    zf = jnp.zeros((L,), jnp.float32)
    nf = jnp.float32(N)

    def chunk_a(k, _):
        off = off_base + k * CHUNK
        hs = [
            pltpu.async_copy(
                logits_hbm.at[pl.ds((b * C + j) * HW + off, CHUNK)],
                buf19.at[pl.ds(j * CHUNK, CHUNK)], sem)
            for j in range(C)
        ]
        for h in hs:
            h.wait()

        def vreg_a(v, _):
            m = buf19[pl.ds(v * L, L)]
            for j in range(1, C):
                m = jnp.maximum(m, buf19[pl.ds(j * CHUNK + v * L, L)])
            den = zf
            for j in range(C):
                den = den + jnp.exp(buf19[pl.ds(j * CHUNK + v * L, L)] - m)
            sbuf[pl.ds(v * L, L)] = 1.0 / den
            mbuf[pl.ds(v * L, L)] = m
            return 0

        lax.fori_loop(0, VPC, vreg_a, 0, unroll=2)
        p0 = p_base + k * CHUNK
        pltpu.sync_copy(sbuf, s_hbm.at[pl.ds(p0, CHUNK)])
        pltpu.sync_copy(mbuf, m_hbm.at[pl.ds(p0, CHUNK)])
        return 0

    lax.fori_loop(0, NCHUNK, chunk_a, 0)

    def class_step(ki, acc):
        c = jnp.where(ci == 0, ki, CLS_PER_CORE + ki)

        def zero_h(i, _):
            hist[pl.ds(i * L, L)] = jnp.zeros((L,), jnp.int32)
            return 0
        lax.fori_loop(0, M // L, zero_h, 0, unroll=4)

        def chunk_b(k, _):
            off = off_base + k * CHUNK
            p0 = p_base + k * CHUNK
            hs = [
                pltpu.async_copy(
                    logits_hbm.at[pl.ds((b * C + c) * HW + off, CHUNK)],
                    lbuf, sem),
                pltpu.async_copy(targets_hbm.at[pl.ds(p0, CHUNK)], tbuf, sem),
                pltpu.async_copy(s_hbm.at[pl.ds(p0, CHUNK)], sbuf, sem),
                pltpu.async_copy(m_hbm.at[pl.ds(p0, CHUNK)], mbuf, sem),
            ]
            for h in hs:
                h.wait()

            def vreg_b(v, _):
                z = lbuf[pl.ds(v * L, L)] - mbuf[pl.ds(v * L, L)]
                p = jnp.exp(jnp.minimum(z, 1.0)) * sbuf[pl.ds(v * L, L)]
                fgm = tbuf[pl.ds(v * L, L)] == c
                e = jnp.abs(jnp.where(fgm, 1.0, 0.0) - p)
                bk = jnp.minimum((e * jnp.float32(M)).astype(jnp.int32), M - 1)
                val = jnp.where(fgm, 65537, 65536).astype(jnp.int32)
                plsc.addupdate_scatter(hist, [bk], val)
                return 0

            lax.fori_loop(0, VPC, vreg_b, 0, unroll=2)
            return 0

        lax.fori_loop(0, NCHUNK, chunk_b, 0)

        pltpu.sync_copy(hist, slots_sh.at[pl.ds(si * M, M)])
        plsc.subcore_barrier()

        base_bkt = si * MB

        hs = [
            pltpu.async_copy(
                slots_sh.at[pl.ds(t * M + base_bkt, MB)],
                slot16.at[pl.ds(t * MB, MB)], sem)
            for t in range(NS)
        ]
        for h in hs:
            h.wait()

        def merge_tile(t, carry):
            def vreg_m(v, cr):
                tc, tf = cr
                u = slot16[pl.ds(t * MB + v * L, L)]
                cnt = ((u >> 16) & 0xFFFF).astype(jnp.float32)
                fgc = (u & 0xFFFF).astype(jnp.float32)
                acnt[pl.ds(v * L, L)] = (
                    jnp.where(t == 0, 0.0, acnt[pl.ds(v * L, L)]) + cnt)
                afg[pl.ds(v * L, L)] = (
                    jnp.where(t == 0, 0.0, afg[pl.ds(v * L, L)]) + fgc)
                return (tc + jnp.sum(cnt), tf + jnp.sum(fgc))

            return lax.fori_loop(0, MB // L, vreg_m, carry, unroll=2)

        tot_c, tot_f = lax.fori_loop(
            0, NS, merge_tile, (jnp.float32(0.0), jnp.float32(0.0)))

        accbuf[...] = jnp.where(lanes == 0, tot_c,
                                jnp.where(lanes == 1, tot_f, 0.0))
        pltpu.sync_copy(accbuf, comm_sh.at[pl.ds(si * L, L)])
        plsc.subcore_barrier()

        pltpu.sync_copy(comm_sh, commbuf)
        p_tot = jnp.float32(0.0)
        k_above = jnp.float32(0.0)
        f_above = jnp.float32(0.0)
        for t in range(NS):
            row = commbuf[pl.ds(t * L, L)]
            tc = jnp.sum(jnp.where(lanes == 0, row, 0.0))
            tf = jnp.sum(jnp.where(lanes == 1, row, 0.0))
            p_tot = p_tot + tf
            gt = jnp.where(t > si, 1.0, 0.0)
            k_above = k_above + gt * tc
            f_above = f_above + gt * tf

        def vreg_s(i, carry):
            kc, fc, ls = carry
            v = MB // L - 1 - i
            cnt = acnt[pl.ds(v * L, L)]
            fgc = afg[pl.ds(v * L, L)]
            tot_cv = jnp.sum(cnt)
            tot_fv = jnp.sum(fgc)
            kb = kc + tot_cv - plsc.cumsum(cnt)
            fb = fc + tot_fv - plsc.cumsum(fgc)
            bb = kb - fb
            j = base_bkt + v * L + lanes
            jf = j.astype(jnp.float32)
            w = (jf + 0.5) * jnp.float32(1.0 / M)
            wprev = jnp.where(j + 1 >= M, 0.0, (jf + 1.5) * jnp.float32(1.0 / M))
            den = jnp.maximum(p_tot + bb, 1.0)
            term = jnp.where(kb < nf, (w - wprev) * (kb + 1.0) / den, 0.0)
            return (kc + tot_cv, fc + tot_fv, ls + term)

        _, _, lsum = lax.fori_loop(
            0, MB // L, vreg_s, (k_above, f_above, zf), unroll=2)

        acc = acc + jnp.where(p_tot > 0.0, lsum, zf)
        plsc.subcore_barrier()
        return acc

    acc = lax.fori_loop(0, CLS_PER_CORE, class_step, zf)
    accbuf[...] = acc
    pltpu.sync_copy(accbuf, out_hbm.at[pl.ds((ci * NS + si) * L, L)])


@functools.partial(jax.jit, static_argnames=())
def kernel(logits, targets):
    logits1d = logits.reshape(4 * C * HW)
    targets1d = targets.reshape(N)
    mesh = plsc.VectorSubcoreMesh(
        core_axis_name="c", subcore_axis_name="s",
        num_cores=NC, num_subcores=NS)
    run = pl.kernel(
        _body,
        out_type=(jax.ShapeDtypeStruct((NC * NS * L,), jnp.float32),
                  jax.ShapeDtypeStruct((N,), jnp.float32),
                  jax.ShapeDtypeStruct((N,), jnp.float32)),
        mesh=mesh,
        compiler_params=pltpu.CompilerParams(needs_layout_passes=False),
        scratch_types=[
            pltpu.VMEM((C * CHUNK,), jnp.float32),
            pltpu.VMEM((CHUNK,), jnp.float32),
            pltpu.VMEM((CHUNK,), jnp.int32),
            pltpu.VMEM((CHUNK,), jnp.float32),
            pltpu.VMEM((CHUNK,), jnp.float32),
            pltpu.VMEM((M,), jnp.int32),
            pltpu.VMEM((NS * MB,), jnp.int32),
            pltpu.VMEM((MB,), jnp.float32),
            pltpu.VMEM((MB,), jnp.float32),
            pltpu.VMEM((NS * L,), jnp.float32),
            pltpu.VMEM((L,), jnp.float32),
            pltpu.SemaphoreType.DMA,
            pltpu.VMEM_SHARED((NS * M,), jnp.int32),
            pltpu.VMEM_SHARED((NS * L,), jnp.float32),
        ],
    )
    partials, _, _ = run(logits1d, targets1d)
    return jnp.sum(partials)


# speedup vs baseline: 23.3592x; 1.7154x over previous
import functools

import jax
import jax.numpy as jnp
from jax import lax
from jax.experimental import pallas as pl
from jax.experimental.pallas import tpu as pltpu
from jax.experimental.pallas import tpu_sc as plsc

NC = 2
NS = 16
L = 16
C = 19
N = 4 * 384 * 384
HW = 384 * 384
M = 8192
PIX_PER_TILE = N // NS
CHUNK = 1024
NCHUNK = PIX_PER_TILE // CHUNK
VPC = CHUNK // L
MB = M // NS
CLS_PER_CORE = 10


def _body(logits_hbm, targets_hbm, out_hbm, s_hbm, m_hbm,
          buf19, lbuf, tbuf, sbuf, mbuf, hist, slot16, acnt, afg,
          commbuf, accbuf, semA, semB, semWA, semWB, slots_sh, comm_sh):
    ci = lax.axis_index("c")
    si = lax.axis_index("s")
    p_base = si * PIX_PER_TILE
    b = si // 4
    off_base = (si % 4) * PIX_PER_TILE

    lanes = lax.iota(jnp.int32, L)
    zf = jnp.zeros((L,), jnp.float32)
    nf = jnp.float32(N)
    rsems = (semA, semB)
    wsems = (semWA, semWB)

    def a_copies(k, h):
        off = off_base + k * CHUNK
        return [
            (logits_hbm.at[pl.ds((b * C + j) * HW + off, CHUNK)],
             buf19.at[pl.ds((h * C + j) * CHUNK, CHUNK)])
            for j in range(C)
        ]

    def a_issue(k, h):
        for src, dst in a_copies(k, h):
            pltpu.async_copy(src, dst, rsems[h])

    def a_wait(k, h):
        for src, dst in a_copies(k, h):
            pltpu.make_async_copy(src, dst, rsems[h]).wait()

    def aw_copies(k, h):
        p0 = p_base + k * CHUNK
        return [
            (sbuf.at[pl.ds(h * CHUNK, CHUNK)], s_hbm.at[pl.ds(p0, CHUNK)]),
            (mbuf.at[pl.ds(h * CHUNK, CHUNK)], m_hbm.at[pl.ds(p0, CHUNK)]),
        ]

    def aw_issue(k, h):
        for src, dst in aw_copies(k, h):
            pltpu.async_copy(src, dst, wsems[h])

    def aw_wait(k, h):
        for src, dst in aw_copies(k, h):
            pltpu.make_async_copy(src, dst, wsems[h]).wait()

    def a_compute(k, h):
        a_wait(k, h)

        @pl.when(k >= 2)
        def _():
            aw_wait(k - 2, h)

        def vreg_a(v, _):
            base = h * C * CHUNK
            m = buf19[pl.ds(base + v * L, L)]
            for j in range(1, C):
                m = jnp.maximum(m, buf19[pl.ds(base + j * CHUNK + v * L, L)])
            den = zf
            for j in range(C):
                den = den + jnp.exp(buf19[pl.ds(base + j * CHUNK + v * L, L)] - m)
            sbuf[pl.ds(h * CHUNK + v * L, L)] = 1.0 / den
            mbuf[pl.ds(h * CHUNK + v * L, L)] = m
            return 0

        lax.fori_loop(0, VPC, vreg_a, 0, unroll=2)
        aw_issue(k, h)

    a_issue(0, 0)

    def pair_a(q, _):
        k0 = 2 * q
        a_issue(k0 + 1, 1)
        a_compute(k0, 0)

        @pl.when(k0 + 2 < NCHUNK)
        def _():
            a_issue(k0 + 2, 0)

        a_compute(k0 + 1, 1)
        return 0

    lax.fori_loop(0, NCHUNK // 2, pair_a, 0)
    aw_wait(NCHUNK - 2, 0)
    aw_wait(NCHUNK - 1, 1)

    def class_step(ki, acc):
        c = jnp.where(ci == 0, ki, CLS_PER_CORE + ki)

        def zero_h(i, _):
            hist[pl.ds(i * L, L)] = jnp.zeros((L,), jnp.int32)
            return 0
        lax.fori_loop(0, M // L, zero_h, 0, unroll=4)

        def b_copies(k, h):
            off = off_base + k * CHUNK
            p0 = p_base + k * CHUNK
            hh = pl.ds(h * CHUNK, CHUNK)
            return [
                (logits_hbm.at[pl.ds((b * C + c) * HW + off, CHUNK)],
                 lbuf.at[hh]),
                (targets_hbm.at[pl.ds(p0, CHUNK)], tbuf.at[hh]),
                (s_hbm.at[pl.ds(p0, CHUNK)], sbuf.at[hh]),
                (m_hbm.at[pl.ds(p0, CHUNK)], mbuf.at[hh]),
            ]

        def b_issue(k, h):
            for src, dst in b_copies(k, h):
                pltpu.async_copy(src, dst, rsems[h])

        def b_compute(k, h):
            for src, dst in b_copies(k, h):
                pltpu.make_async_copy(src, dst, rsems[h]).wait()

            def vreg_b(v, _):
                hv = h * CHUNK + v * L
                z = lbuf[pl.ds(hv, L)] - mbuf[pl.ds(hv, L)]
                p = jnp.exp(jnp.minimum(z, 1.0)) * sbuf[pl.ds(hv, L)]
                fgm = tbuf[pl.ds(hv, L)] == c
                e = jnp.abs(jnp.where(fgm, 1.0, 0.0) - p)
                bk = jnp.minimum((e * jnp.float32(M)).astype(jnp.int32), M - 1)
                val = jnp.where(fgm, 65537, 65536).astype(jnp.int32)
                plsc.addupdate_scatter(hist, [bk], val)
                return 0

            lax.fori_loop(0, VPC, vreg_b, 0, unroll=2)

        b_issue(0, 0)

        def pair_b(q, _):
            k0 = 2 * q
            b_issue(k0 + 1, 1)
            b_compute(k0, 0)

            @pl.when(k0 + 2 < NCHUNK)
            def _():
                b_issue(k0 + 2, 0)

            b_compute(k0 + 1, 1)
            return 0

        lax.fori_loop(0, NCHUNK // 2, pair_b, 0)

        pltpu.sync_copy(hist, slots_sh.at[pl.ds(si * M, M)])
        plsc.subcore_barrier()

        base_bkt = si * MB

        hs = [
            pltpu.async_copy(
                slots_sh.at[pl.ds(t * M + base_bkt, MB)],
                slot16.at[pl.ds(t * MB, MB)], semA)
            for t in range(NS)
        ]
        for h in hs:
            h.wait()

        def merge_tile(t, carry):
            def vreg_m(v, cr):
                tc, tf = cr
                u = slot16[pl.ds(t * MB + v * L, L)]
                cnt = ((u >> 16) & 0xFFFF).astype(jnp.float32)
                fgc = (u & 0xFFFF).astype(jnp.float32)
                acnt[pl.ds(v * L, L)] = (
                    jnp.where(t == 0, 0.0, acnt[pl.ds(v * L, L)]) + cnt)
                afg[pl.ds(v * L, L)] = (
                    jnp.where(t == 0, 0.0, afg[pl.ds(v * L, L)]) + fgc)
                return (tc + jnp.sum(cnt), tf + jnp.sum(fgc))

            return lax.fori_loop(0, MB // L, vreg_m, carry, unroll=2)

        tot_c, tot_f = lax.fori_loop(
            0, NS, merge_tile, (jnp.float32(0.0), jnp.float32(0.0)))

        accbuf[...] = jnp.where(lanes == 0, tot_c,
                                jnp.where(lanes == 1, tot_f, 0.0))
        pltpu.sync_copy(accbuf, comm_sh.at[pl.ds(si * L, L)])
        plsc.subcore_barrier()

        pltpu.sync_copy(comm_sh, commbuf)
        p_tot = jnp.float32(0.0)
        k_above = jnp.float32(0.0)
        f_above = jnp.float32(0.0)
        for t in range(NS):
            row = commbuf[pl.ds(t * L, L)]
            tc = jnp.sum(jnp.where(lanes == 0, row, 0.0))
            tf = jnp.sum(jnp.where(lanes == 1, row, 0.0))
            p_tot = p_tot + tf
            gt = jnp.where(t > si, 1.0, 0.0)
            k_above = k_above + gt * tc
            f_above = f_above + gt * tf

        def vreg_s(i, carry):
            kc, fc, ls = carry
            v = MB // L - 1 - i
            cnt = acnt[pl.ds(v * L, L)]
            fgc = afg[pl.ds(v * L, L)]
            tot_cv = jnp.sum(cnt)
            tot_fv = jnp.sum(fgc)
            kb = kc + tot_cv - plsc.cumsum(cnt)
            fb = fc + tot_fv - plsc.cumsum(fgc)
            bb = kb - fb
            j = base_bkt + v * L + lanes
            jf = j.astype(jnp.float32)
            w = (jf + 0.5) * jnp.float32(1.0 / M)
            wprev = jnp.where(j + 1 >= M, 0.0, (jf + 1.5) * jnp.float32(1.0 / M))
            den = jnp.maximum(p_tot + bb, 1.0)
            term = jnp.where(kb < nf, (w - wprev) * (kb + 1.0) / den, 0.0)
            return (kc + tot_cv, fc + tot_fv, ls + term)

        _, _, lsum = lax.fori_loop(
            0, MB // L, vreg_s, (k_above, f_above, zf), unroll=2)

        acc = acc + jnp.where(p_tot > 0.0, lsum, zf)
        plsc.subcore_barrier()
        return acc

    acc = lax.fori_loop(0, CLS_PER_CORE, class_step, zf)
    accbuf[...] = acc
    pltpu.sync_copy(accbuf, out_hbm.at[pl.ds((ci * NS + si) * L, L)])


@functools.partial(jax.jit, static_argnames=())
def kernel(logits, targets):
    logits1d = logits.reshape(4 * C * HW)
    targets1d = targets.reshape(N)
    mesh = plsc.VectorSubcoreMesh(
        core_axis_name="c", subcore_axis_name="s",
        num_cores=NC, num_subcores=NS)
    run = pl.kernel(
        _body,
        out_type=(jax.ShapeDtypeStruct((NC * NS * L,), jnp.float32),
                  jax.ShapeDtypeStruct((N,), jnp.float32),
                  jax.ShapeDtypeStruct((N,), jnp.float32)),
        mesh=mesh,
        compiler_params=pltpu.CompilerParams(needs_layout_passes=False),
        scratch_types=[
            pltpu.VMEM((2 * C * CHUNK,), jnp.float32),
            pltpu.VMEM((2 * CHUNK,), jnp.float32),
            pltpu.VMEM((2 * CHUNK,), jnp.int32),
            pltpu.VMEM((2 * CHUNK,), jnp.float32),
            pltpu.VMEM((2 * CHUNK,), jnp.float32),
            pltpu.VMEM((M,), jnp.int32),
            pltpu.VMEM((NS * MB,), jnp.int32),
            pltpu.VMEM((MB,), jnp.float32),
            pltpu.VMEM((MB,), jnp.float32),
            pltpu.VMEM((NS * L,), jnp.float32),
            pltpu.VMEM((L,), jnp.float32),
            pltpu.SemaphoreType.DMA,
            pltpu.SemaphoreType.DMA,
            pltpu.SemaphoreType.DMA,
            pltpu.SemaphoreType.DMA,
            pltpu.VMEM_SHARED((NS * M,), jnp.int32),
            pltpu.VMEM_SHARED((NS * L,), jnp.float32),
        ],
    )
    partials, _, _ = run(logits1d, targets1d)
    return jnp.sum(partials)
